# baseline (speedup 1.0000x reference)
import functools

import jax
import jax.numpy as jnp
from jax import lax
from jax.experimental import pallas as pl
from jax.experimental.pallas import tpu as pltpu
from jax.experimental.pallas import tpu_sc as plsc

_D = 64
_NC = 2
_NS = 16
_NW = _NC * _NS
_BLK = 128
_L = 16
_NBUF = 5
_NOBUF = 2


def _make_lookup(b1: int, b2: int, vocab: int):
    mesh = plsc.VectorSubcoreMesh(core_axis_name="c", subcore_axis_name="s")

    @functools.partial(
        pl.kernel,
        out_type=jax.ShapeDtypeStruct((b2, _D, b1), jnp.float32),
        mesh=mesh,
        scratch_types=[
            pltpu.VMEM((b2, _BLK), jnp.int32),
            [pltpu.VMEM((_BLK,), jnp.int32) for _ in range(_NBUF)],
            [pltpu.VMEM((_BLK, 2 * _D), jnp.float32) for _ in range(_NBUF)],
            [pltpu.VMEM((_D, _BLK), jnp.float32) for _ in range(_NOBUF)],
            [pltpu.SemaphoreType.DMA for _ in range(_NBUF)],
            [pltpu.SemaphoreType.DMA for _ in range(_NOBUF)],
        ],
        compiler_params=pltpu.CompilerParams(
            use_tc_tiling_on_sc=True, needs_layout_passes=False),
    )
    def lookup(x_hbm, lut_hbm, out_hbm, idx_all, pidx_v, rows_v, out_v, gsem,
               wsem):
        wid = lax.axis_index("s") * _NC + lax.axis_index("c")
        col0 = wid * _BLK
        lane = lax.iota(jnp.int32, _L)

        pltpu.sync_copy(x_hbm.at[:, pl.ds(col0, _BLK)], idx_all)

        def start_gather(t, b):
            for g in range(_BLK // _L):
                sl = pl.ds(g * _L, _L)
                pidx_v[b][sl] = lax.shift_right_logical(idx_all[t, sl], 1)
            pltpu.async_copy(lut_hbm.at[pidx_v[b]], rows_v[b], gsem[b])

        def process(t, b, ob):
            pltpu.make_async_copy(
                lut_hbm.at[pidx_v[b]], rows_v[b], gsem[b]).wait()
            for g in range(_BLK // _L):
                rows16 = lane + (g * _L)
                half16 = (idx_all[t, pl.ds(g * _L, _L)] & 1) * _D

                @plsc.parallel_loop(0, _D, 1, unroll=8)
                def _(d):
                    rot = (rows16 + d) & (_D - 1)
                    vals = plsc.load_gather(rows_v[b], [rows16, half16 + rot])
                    plsc.store_scatter(out_v[ob], [rot, rows16], vals * 8.0)
            pltpu.async_copy(out_v[ob], out_hbm.at[t, :, pl.ds(col0, _BLK)],
                             wsem[ob])

        def wait_write(t, ob):
            pltpu.make_async_copy(
                out_v[ob], out_hbm.at[t, :, pl.ds(col0, _BLK)],
                wsem[ob]).wait()

        for b in range(_NBUF):
            start_gather(b, b)

        def step_quad(tq, carry):
            t = _NBUF * tq
            for b in range(_NBUF):
                ob = b % _NOBUF

                @pl.when(t + b >= _NOBUF)
                def _():
                    wait_write(t + b, ob)

                process(t + b, b, ob)

                @pl.when(tq < b2 // _NBUF - 1)
                def _():
                    start_gather(t + b + _NBUF, b)

            return carry

        lax.fori_loop(0, b2 // _NBUF, step_quad, 0)
        wait_write(b2 - 2, 0)
        wait_write(b2 - 1, 1)

    return lookup


def _make_repack(vocab: int):
    nblk = vocab // _BLK
    mesh = plsc.VectorSubcoreMesh(core_axis_name="c", subcore_axis_name="s")
    nbuf = 4

    @functools.partial(
        pl.kernel,
        out_type=jax.ShapeDtypeStruct((vocab // 2, 2 * _D), jnp.float32),
        mesh=mesh,
        scratch_types=[
            [pltpu.VMEM((_D, _BLK), jnp.float32) for _ in range(nbuf)],
            [pltpu.VMEM((_D, 2 * _D), jnp.float32) for _ in range(nbuf)],
            pltpu.VMEM(((vocab % _BLK) // 2, 2 * _D), jnp.float32),
            [pltpu.SemaphoreType.DMA for _ in range(nbuf)],
            [pltpu.SemaphoreType.DMA for _ in range(nbuf)],
        ],
        compiler_params=pltpu.CompilerParams(
            use_tc_tiling_on_sc=True, needs_layout_passes=False),
    )
    def repack(lutt_hbm, tail_hbm, w_hbm, tile_v, wblk_v, tail_v, rsem, wsem):
        wid = lax.axis_index("s") * _NC + lax.axis_index("c")
        lane = lax.iota(jnp.int32, _L)
        nstep = nblk // _NW + 1

        def valid(k):
            return wid + _NW * k < nblk

        def col_of(k):
            return pl.multiple_of((wid + _NW * k) * _BLK, _BLK)

        def row_of(k):
            return pl.multiple_of((wid + _NW * k) * (_BLK // 2), 8)

        def start_read(k, b):
            pltpu.async_copy(
                lutt_hbm.at[:, pl.ds(col_of(k), _BLK)], tile_v[b], rsem[b])

        def process(k, b):
            pltpu.make_async_copy(
                lutt_hbm.at[:, pl.ds(col_of(k), _BLK)], tile_v[b],
                rsem[b]).wait()
            for gw in range(_D // _L):
                w16 = lane + gw * _L
                for p in range(2):
                    src_col = 2 * w16 + p

                    @plsc.parallel_loop(0, _D, 1, unroll=8)
                    def _(d):
                        rot = (w16 + d) & (_D - 1)
                        vals = plsc.load_gather(tile_v[b], [rot, src_col])
                        plsc.store_scatter(wblk_v[b], [w16, rot + p * _D],
                                           vals)
            pltpu.async_copy(
                wblk_v[b], w_hbm.at[pl.ds(row_of(k), _D)], wsem[b])

        def wait_write(k, b):
            pltpu.make_async_copy(
                wblk_v[b], w_hbm.at[pl.ds(row_of(k), _D)],
                wsem[b]).wait()

        for b in range(nbuf):
            start_read(b, b)

        def step_pair(k2, carry):
            k = nbuf * k2
            for b in range(nbuf):

                @pl.when(valid(k + b))
                def _():

                    @pl.when(k2 > 0)
                    def _():
                        wait_write(k + b, b)

                    process(k + b, b)

                    @pl.when(valid(k + b + nbuf))
                    def _():
                        start_read(k + b + nbuf, b)

            return carry

        lax.fori_loop(0, (nstep + nbuf - 1) // nbuf, step_pair, 0)
        for b in range(nbuf):
            wait_write(nstep - nbuf + b, b)

        @pl.when(wid == 0)
        def _():
            pltpu.sync_copy(tail_hbm, tail_v)
            pltpu.sync_copy(
                tail_v, w_hbm.at[pl.ds(nblk * _BLK // 2, (vocab % _BLK) // 2)])

    return repack


def kernel(x, lut):
    b1, b2 = x.shape
    vocab = lut.shape[0]
    xt = x.T
    lutt = lut.T
    ntail = vocab % _BLK
    tail = lut[vocab - ntail:].reshape(ntail // 2, 2 * _D)
    lut2 = _make_repack(vocab)(lutt, tail)
    out_t = _make_lookup(b1, b2, vocab)(xt, lut2)
    return out_t.transpose(2, 0, 1)


# speedup vs baseline: 1.0094x; 1.0094x over previous
import functools

import jax
import jax.numpy as jnp
from jax import lax
from jax.experimental import pallas as pl
from jax.experimental.pallas import tpu as pltpu
from jax.experimental.pallas import tpu_sc as plsc

_D = 64
_NC = 2
_NS = 16
_NW = _NC * _NS
_BLK = 128
_L = 16
_NBUF = 4
_NOBUF = 2


def _make_lookup(b1: int, b2: int, vocab: int):
    mesh = plsc.VectorSubcoreMesh(core_axis_name="c", subcore_axis_name="s")

    @functools.partial(
        pl.kernel,
        out_type=jax.ShapeDtypeStruct((b2, _D, b1), jnp.float32),
        mesh=mesh,
        scratch_types=[
            pltpu.VMEM((b2, _BLK), jnp.int32),
            [pltpu.VMEM((_BLK,), jnp.int32) for _ in range(_NBUF)],
            [pltpu.VMEM((_BLK, 2 * _D), jnp.float32) for _ in range(_NBUF)],
            [pltpu.VMEM((_D, _BLK), jnp.float32) for _ in range(_NOBUF)],
            [pltpu.SemaphoreType.DMA for _ in range(_NBUF)],
            [pltpu.SemaphoreType.DMA for _ in range(_NOBUF)],
        ],
        compiler_params=pltpu.CompilerParams(
            use_tc_tiling_on_sc=True, needs_layout_passes=False),
    )
    def lookup(x_hbm, lut_hbm, out_hbm, idx_all, pidx_v, rows_v, out_v, gsem,
               wsem):
        wid = lax.axis_index("s") * _NC + lax.axis_index("c")
        col0 = wid * _BLK
        lane = lax.iota(jnp.int32, _L)

        pltpu.sync_copy(x_hbm.at[:, pl.ds(col0, _BLK)], idx_all)

        def start_gather(t, b):
            for g in range(_BLK // _L):
                sl = pl.ds(g * _L, _L)
                pidx_v[b][sl] = lax.shift_right_logical(idx_all[t, sl], 1)
            pltpu.async_copy(lut_hbm.at[pidx_v[b]], rows_v[b], gsem[b])

        def process(t, b, ob):
            pltpu.make_async_copy(
                lut_hbm.at[pidx_v[b]], rows_v[b], gsem[b]).wait()
            for g in range(_BLK // _L):
                rows16 = lane + (g * _L)
                half16 = (idx_all[t, pl.ds(g * _L, _L)] & 1) * _D

                @plsc.parallel_loop(0, _D, 1, unroll=8)
                def _(d):
                    rot = (rows16 + d) & (_D - 1)
                    vals = plsc.load_gather(rows_v[b], [rows16, half16 + rot])
                    plsc.store_scatter(out_v[ob], [rot, rows16], vals * 8.0)
            pltpu.async_copy(out_v[ob], out_hbm.at[t, :, pl.ds(col0, _BLK)],
                             wsem[ob])

        def wait_write(t, ob):
            pltpu.make_async_copy(
                out_v[ob], out_hbm.at[t, :, pl.ds(col0, _BLK)],
                wsem[ob]).wait()

        for b in range(_NBUF):
            start_gather(b, b)

        def step_quad(tq, carry):
            t = _NBUF * tq
            for b in range(_NBUF):
                ob = b % _NOBUF

                @pl.when(t + b >= _NOBUF)
                def _():
                    wait_write(t + b, ob)

                process(t + b, b, ob)

                @pl.when(tq < b2 // _NBUF - 1)
                def _():
                    start_gather(t + b + _NBUF, b)

            return carry

        lax.fori_loop(0, b2 // _NBUF, step_quad, 0)
        wait_write(b2 - 2, 0)
        wait_write(b2 - 1, 1)

    return lookup


def _make_repack(vocab: int):
    nblk = vocab // _BLK
    mesh = plsc.VectorSubcoreMesh(core_axis_name="c", subcore_axis_name="s")
    nbuf = 4

    @functools.partial(
        pl.kernel,
        out_type=jax.ShapeDtypeStruct((vocab // 2, 2 * _D), jnp.float32),
        mesh=mesh,
        scratch_types=[
            [pltpu.VMEM((_D, _BLK), jnp.float32) for _ in range(nbuf)],
            [pltpu.VMEM((_D, 2 * _D), jnp.float32) for _ in range(nbuf)],
            pltpu.VMEM(((vocab % _BLK) // 2, 2 * _D), jnp.float32),
            [pltpu.SemaphoreType.DMA for _ in range(nbuf)],
            [pltpu.SemaphoreType.DMA for _ in range(nbuf)],
        ],
        compiler_params=pltpu.CompilerParams(
            use_tc_tiling_on_sc=True, needs_layout_passes=False),
    )
    def repack(lutt_hbm, tail_hbm, w_hbm, tile_v, wblk_v, tail_v, rsem, wsem):
        wid = lax.axis_index("s") * _NC + lax.axis_index("c")
        lane = lax.iota(jnp.int32, _L)
        nstep = nblk // _NW + 1

        def valid(k):
            return wid + _NW * k < nblk

        def col_of(k):
            return pl.multiple_of((wid + _NW * k) * _BLK, _BLK)

        def row_of(k):
            return pl.multiple_of((wid + _NW * k) * (_BLK // 2), 8)

        def start_read(k, b):
            pltpu.async_copy(
                lutt_hbm.at[:, pl.ds(col_of(k), _BLK)], tile_v[b], rsem[b])

        def process(k, b):
            pltpu.make_async_copy(
                lutt_hbm.at[:, pl.ds(col_of(k), _BLK)], tile_v[b],
                rsem[b]).wait()
            for gw in range(_D // _L):
                w16 = lane + gw * _L
                for p in range(2):
                    src_col = 2 * w16 + p

                    @plsc.parallel_loop(0, _D, 1, unroll=8)
                    def _(d):
                        rot = (w16 + d) & (_D - 1)
                        vals = plsc.load_gather(tile_v[b], [rot, src_col])
                        plsc.store_scatter(wblk_v[b], [w16, rot + p * _D],
                                           vals)
            pltpu.async_copy(
                wblk_v[b], w_hbm.at[pl.ds(row_of(k), _D)], wsem[b])

        def wait_write(k, b):
            pltpu.make_async_copy(
                wblk_v[b], w_hbm.at[pl.ds(row_of(k), _D)],
                wsem[b]).wait()

        for b in range(nbuf):
            start_read(b, b)

        def step_pair(k2, carry):
            k = nbuf * k2
            for b in range(nbuf):

                @pl.when(valid(k + b))
                def _():

                    @pl.when(k2 > 0)
                    def _():
                        wait_write(k + b, b)

                    process(k + b, b)

                    @pl.when(valid(k + b + nbuf))
                    def _():
                        start_read(k + b + nbuf, b)

            return carry

        lax.fori_loop(0, (nstep + nbuf - 1) // nbuf, step_pair, 0)
        for b in range(nbuf):
            wait_write(nstep - nbuf + b, b)

        @pl.when(wid == 0)
        def _():
            pltpu.sync_copy(tail_hbm, tail_v)
            pltpu.sync_copy(
                tail_v, w_hbm.at[pl.ds(nblk * _BLK // 2, (vocab % _BLK) // 2)])

    return repack


def kernel(x, lut):
    b1, b2 = x.shape
    vocab = lut.shape[0]
    xt = x.T
    lutt = lut.T
    ntail = vocab % _BLK
    tail = lut[vocab - ntail:].reshape(ntail // 2, 2 * _D)
    lut2 = _make_repack(vocab)(lutt, tail)
    out_t = _make_lookup(b1, b2, vocab)(xt, lut2)
    return out_t.transpose(2, 0, 1)
